# baseline (speedup 1.0000x reference)
import functools

import jax
import jax.numpy as jnp
from jax import lax
from jax.experimental import pallas as pl
from jax.experimental.pallas import tpu as pltpu

_NUM_LAYERS = 3
_TOP_K = 10
_MIN_GATE = 0.1
_B = 128
_D = 100000

_TPS = 338
_SPB = 8
_W = _TPS * 8 * _SPB
_GE = (_D + _W - 1) // _W
_NTILES = _D // 8

_TILE = 2048
_NT = (_D + _TILE - 1) // _TILE


def _ent_body(x_ref, out_ref):
    pid = pl.program_id(0)

    @pl.when(pid == 0)
    def _():
        out_ref[...] = jnp.zeros_like(out_ref)

    p = x_ref[...]
    col = pid * _W + lax.broadcasted_iota(jnp.int32, (_B, _W), 1)
    t = jnp.where(col < _D, p * jnp.log(p + 1e-10), 0.0)

    cache = {}

    def tc(c):
        if c not in cache:
            cache[c] = jnp.transpose(t[:, c * 128 : (c + 1) * 128])
        return cache[c]

    total = out_ref[...]
    for g in range(_SPB):
        acc = None
        for k in range(_TPS):
            kg = g * _TPS + k
            c, r = kg // 16, kg % 16
            tile = tc(c)[8 * r : 8 * r + 8, :]
            acc = tile if acc is None else acc + tile
        b4 = acc[0:4, :] + acc[4:8, :]
        b2 = b4[0:2, :] + b4[2:4, :]
        b1 = b2[0:1, :] + b2[1:2, :]
        total = total + b1
    out_ref[...] = jnp.where(pid == _GE - 1, -total, total)


def _entropy(lw):
    return pl.pallas_call(
        _ent_body,
        grid=(_GE,),
        in_specs=[pl.BlockSpec((_B, _W), lambda i: (0, i))],
        out_specs=pl.BlockSpec((1, _B), lambda i: (0, 0)),
        out_shape=jax.ShapeDtypeStruct((1, _B), jnp.float32),
        compiler_params=pltpu.CompilerParams(
            dimension_semantics=("arbitrary",)
        ),
    )(lw)


def _gate_body(ent_ref, w1t_ref, b1_ref, w2t_ref, b2_ref, gate_ref):
    hidden = _NUM_LAYERS * 4
    h = jnp.maximum(
        jnp.dot(ent_ref[...], w1t_ref[...], preferred_element_type=jnp.float32)
        + b1_ref[0:1, :],
        0.0,
    )
    gl = (
        jnp.dot(h, w2t_ref[...], preferred_element_type=jnp.float32)
        + b2_ref[0:1, :]
    )
    g = [gl[:, i : i + 1] for i in range(_NUM_LAYERS)]
    m = jnp.maximum(jnp.maximum(g[0], g[1]), g[2])
    e = [jnp.exp(gi - m) for gi in g]
    s = (e[0] + e[1]) + e[2]
    scale = 1.0 - _NUM_LAYERS * _MIN_GATE
    for i in range(_NUM_LAYERS):
        gate_ref[:, i : i + 1] = (e[i] / s) * scale + _MIN_GATE


def _gate(ent, W1t, b1, W2t, b2):
    hidden = _NUM_LAYERS * 4
    return pl.pallas_call(
        _gate_body,
        in_specs=[
            pl.BlockSpec((_B, _NUM_LAYERS), lambda: (0, 0)),
            pl.BlockSpec((_NUM_LAYERS, hidden), lambda: (0, 0)),
            pl.BlockSpec((1, hidden), lambda: (0, 0)),
            pl.BlockSpec((hidden, _NUM_LAYERS), lambda: (0, 0)),
            pl.BlockSpec((1, _NUM_LAYERS), lambda: (0, 0)),
        ],
        out_specs=pl.BlockSpec((_B, _NUM_LAYERS), lambda: (0, 0)),
        out_shape=jax.ShapeDtypeStruct((_B, _NUM_LAYERS), jnp.float32),
    )(ent, W1t, b1, W2t, b2)


def _combine_body(x0, x1, x2, gate, out_ref):
    g0 = gate[:, 0:1]
    g1 = gate[:, 1:2]
    g2 = gate[:, 2:3]
    out_ref[...] = (g0 * x0[...] + g1 * x1[...] + g2 * x2[...]) * 100.0


def _lw_spec():
    return pl.BlockSpec((_B, _TILE), lambda i: (0, i))


def _compute_logits(lw0, lw1, lw2, gate):
    return pl.pallas_call(
        _combine_body,
        grid=(_NT,),
        in_specs=[
            _lw_spec(),
            _lw_spec(),
            _lw_spec(),
            pl.BlockSpec((_B, _NUM_LAYERS), lambda i: (0, 0)),
        ],
        out_specs=pl.BlockSpec((_B, _TILE), lambda i: (0, i)),
        out_shape=jax.ShapeDtypeStruct((_B, _D), jnp.float32),
        compiler_params=pltpu.CompilerParams(
            dimension_semantics=("arbitrary",)
        ),
    )(lw0, lw1, lw2, gate)


@jax.jit
def kernel(lw0, lw1, lw2, W1, b1, W2, b2):
    ent0 = _entropy(lw0)
    ent1 = _entropy(lw1)
    ent2 = _entropy(lw2)
    ent = jnp.concatenate([ent0, ent1, ent2], axis=0).T
    gate = _gate(ent, W1.T, b1[None, :], W2.T, b2[None, :])
    logits = _compute_logits(lw0, lw1, lw2, gate)
    _, top_indices = jax.lax.top_k(logits, _TOP_K)
    return top_indices, logits, gate
